# baseline (speedup 1.0000x reference)
import functools

import jax
import jax.numpy as jnp
from jax import lax
from jax.experimental import pallas as pl
from jax.experimental.pallas import tpu as pltpu
from jax.experimental.pallas import tpu_sc as plsc

VOCAB = 100000
EMB_DIM = 128
HIDDEN = 512
OUT_DIM = 128
BATCH = 4096
SEQ = 200

NTOK = BATCH * SEQ
NC, NS = 2, 16
NW = NC * NS
IDX_ROWS = NTOK // 128

NCHUNK = 5
CH_TOK = NTOK // NCHUNK
CH_ROWS = IDX_ROWS // NCHUNK
ROWS_PER_W = CH_ROWS // NW
K = 8
HALF = K // 2
GROUPS = ROWS_PER_W // K

TILE = 8192
CH_STEPS = CH_TOK // TILE


@functools.partial(
    pl.kernel,
    mesh=plsc.VectorSubcoreMesh(core_axis_name="c", subcore_axis_name="s"),
    out_type=jax.ShapeDtypeStruct((CH_TOK, EMB_DIM), jnp.float32),
    scratch_types=[
        pltpu.VMEM((K, 128), jnp.int32),
        pltpu.VMEM((HALF * 128, EMB_DIM), jnp.float32),
        pltpu.SemaphoreType.DMA,
    ],
)
def _sc_gather(x2_hbm, emb_hbm, out_hbm, idx_v, rows_v, sem):
    wid = lax.axis_index("s") * NC + lax.axis_index("c")
    row0 = wid * ROWS_PER_W

    def group(g, carry):
        r = row0 + g * K
        pltpu.sync_copy(x2_hbm.at[pl.ds(r, K)], idx_v)
        for w in range(2):
            for j in range(HALF):
                pltpu.async_copy(
                    emb_hbm.at[idx_v.at[w * HALF + j]],
                    rows_v.at[pl.ds(j * 128, 128)], sem,
                )
            for j in range(HALF):
                pltpu.make_async_copy(
                    emb_hbm.at[idx_v.at[w * HALF + j]],
                    rows_v.at[pl.ds(j * 128, 128)], sem,
                ).wait()
            pltpu.sync_copy(
                rows_v, out_hbm.at[pl.ds((r + w * HALF) * 128, HALF * 128)]
            )
        return carry

    lax.fori_loop(0, GROUPS, group, 0)


def _mlp_body(acc_ref, e_ref, w1_ref, b1_ref, w2_ref, b2_ref, w3_ref, b3_ref,
              o_ref):
    del acc_ref
    h = jnp.dot(e_ref[...].astype(jnp.bfloat16), w1_ref[...],
                preferred_element_type=jnp.float32).astype(jnp.bfloat16)
    h = jnp.maximum(h + b1_ref[...], jnp.bfloat16(0.0))
    h = jnp.dot(h, w2_ref[...],
                preferred_element_type=jnp.float32).astype(jnp.bfloat16)
    h = jnp.maximum(h + b2_ref[...], jnp.bfloat16(0.0))
    o = jnp.dot(h, w3_ref[...], preferred_element_type=jnp.float32)
    o_ref[...] = o + b3_ref[...]


def _mlp_chunk_call(chunk):
    step0 = chunk * CH_STEPS
    return pl.pallas_call(
        _mlp_body,
        grid=(CH_STEPS,),
        in_specs=[
            pl.BlockSpec(memory_space=pl.ANY),
            pl.BlockSpec((TILE, EMB_DIM), lambda i: (i, 0)),
            pl.BlockSpec((EMB_DIM, HIDDEN), lambda i: (0, 0)),
            pl.BlockSpec((1, HIDDEN), lambda i: (0, 0)),
            pl.BlockSpec((HIDDEN, HIDDEN), lambda i: (0, 0)),
            pl.BlockSpec((1, HIDDEN), lambda i: (0, 0)),
            pl.BlockSpec((HIDDEN, OUT_DIM), lambda i: (0, 0)),
            pl.BlockSpec((1, OUT_DIM), lambda i: (0, 0)),
        ],
        out_specs=pl.BlockSpec((TILE, OUT_DIM), lambda i: (step0 + i, 0)),
        out_shape=jax.ShapeDtypeStruct((NTOK, OUT_DIM), jnp.float32),
        input_output_aliases={0: 0},
        compiler_params=pltpu.CompilerParams(
            dimension_semantics=("arbitrary",)
        ),
    )


def kernel(x, emb, W1, b1, W2, b2, W3, b3):
    x2 = x.reshape(IDX_ROWS, 128).astype(jnp.int32)
    w_args = (
        W1.astype(jnp.bfloat16), b1.astype(jnp.bfloat16).reshape(1, HIDDEN),
        W2.astype(jnp.bfloat16), b2.astype(jnp.bfloat16).reshape(1, HIDDEN),
        W3.astype(jnp.bfloat16), b3.reshape(1, OUT_DIM),
    )
    es = [
        _sc_gather(lax.slice_in_dim(x2, c * CH_ROWS, (c + 1) * CH_ROWS), emb)
        for c in range(NCHUNK)
    ]
    out = jnp.zeros((NTOK, OUT_DIM), jnp.float32)
    for c in range(NCHUNK):
        out = _mlp_chunk_call(c)(out, es[c], *w_args)
    return out.reshape(BATCH, SEQ, OUT_DIM)


# speedup vs baseline: 1.2992x; 1.2992x over previous
import functools

import jax
import jax.numpy as jnp
from jax import lax
from jax.experimental import pallas as pl
from jax.experimental.pallas import tpu as pltpu
from jax.experimental.pallas import tpu_sc as plsc

VOCAB = 100000
EMB_DIM = 128
HIDDEN = 512
OUT_DIM = 128
BATCH = 4096
SEQ = 200

NTOK = BATCH * SEQ
NC, NS = 2, 16
NW = NC * NS
IDX_ROWS = NTOK // 128

NCHUNK = 5
CH_TOK = NTOK // NCHUNK
CH_ROWS = IDX_ROWS // NCHUNK
ROWS_PER_W = CH_ROWS // NW
K = 8
HALF = K // 2
GROUPS = ROWS_PER_W // K

TILE = 8192
CH_STEPS = CH_TOK // TILE


@functools.partial(
    pl.kernel,
    mesh=plsc.VectorSubcoreMesh(core_axis_name="c", subcore_axis_name="s"),
    out_type=jax.ShapeDtypeStruct((CH_TOK, EMB_DIM), jnp.float32),
    scratch_types=[
        pltpu.VMEM((K, 128), jnp.int32),
        pltpu.VMEM((HALF * 128, EMB_DIM), jnp.float32),
        pltpu.SemaphoreType.DMA,
    ],
)
def _sc_gather(x2_hbm, emb_hbm, out_hbm, idx_v, rows_v, sem):
    wid = lax.axis_index("s") * NC + lax.axis_index("c")
    row0 = wid * ROWS_PER_W

    def group(g, carry):
        r = row0 + g * K
        pltpu.sync_copy(x2_hbm.at[pl.ds(r, K)], idx_v)
        for w in range(2):
            for j in range(HALF):
                pltpu.async_copy(
                    emb_hbm.at[idx_v.at[w * HALF + j]],
                    rows_v.at[pl.ds(j * 128, 128)], sem,
                )
            for j in range(HALF):
                pltpu.make_async_copy(
                    emb_hbm.at[idx_v.at[w * HALF + j]],
                    rows_v.at[pl.ds(j * 128, 128)], sem,
                ).wait()
            pltpu.sync_copy(
                rows_v, out_hbm.at[pl.ds((r + w * HALF) * 128, HALF * 128)]
            )
        return carry

    lax.fori_loop(0, GROUPS, group, 0)


def _mlp_body(*refs):
    if len(refs) == 9:
        refs = refs[1:]
    e_ref, w1_ref, b1_ref, w2_ref, b2_ref, w3_ref, b3_ref, o_ref = refs
    h = jnp.dot(e_ref[...].astype(jnp.bfloat16), w1_ref[...],
                preferred_element_type=jnp.float32).astype(jnp.bfloat16)
    h = jnp.maximum(h + b1_ref[...], jnp.bfloat16(0.0))
    h = jnp.dot(h, w2_ref[...],
                preferred_element_type=jnp.float32).astype(jnp.bfloat16)
    h = jnp.maximum(h + b2_ref[...], jnp.bfloat16(0.0))
    o = jnp.dot(h, w3_ref[...], preferred_element_type=jnp.float32)
    o_ref[...] = o + b3_ref[...]


def _mlp_chunk_call(chunk):
    step0 = chunk * CH_STEPS
    alias = ([pl.BlockSpec(memory_space=pl.ANY)] if chunk else [])
    return pl.pallas_call(
        _mlp_body,
        grid=(CH_STEPS,),
        in_specs=alias + [
            pl.BlockSpec((TILE, EMB_DIM), lambda i: (i, 0)),
            pl.BlockSpec((EMB_DIM, HIDDEN), lambda i: (0, 0)),
            pl.BlockSpec((1, HIDDEN), lambda i: (0, 0)),
            pl.BlockSpec((HIDDEN, HIDDEN), lambda i: (0, 0)),
            pl.BlockSpec((1, HIDDEN), lambda i: (0, 0)),
            pl.BlockSpec((HIDDEN, OUT_DIM), lambda i: (0, 0)),
            pl.BlockSpec((1, OUT_DIM), lambda i: (0, 0)),
        ],
        out_specs=pl.BlockSpec((TILE, OUT_DIM), lambda i: (step0 + i, 0)),
        out_shape=jax.ShapeDtypeStruct((NTOK, OUT_DIM), jnp.float32),
        input_output_aliases={0: 0} if chunk else {},
        compiler_params=pltpu.CompilerParams(
            dimension_semantics=("arbitrary",)
        ),
    )


def kernel(x, emb, W1, b1, W2, b2, W3, b3):
    x2 = x.reshape(IDX_ROWS, 128).astype(jnp.int32)
    w_args = (
        W1.astype(jnp.bfloat16), b1.astype(jnp.bfloat16).reshape(1, HIDDEN),
        W2.astype(jnp.bfloat16), b2.astype(jnp.bfloat16).reshape(1, HIDDEN),
        W3.astype(jnp.bfloat16), b3.reshape(1, OUT_DIM),
    )
    es = [
        _sc_gather(lax.slice_in_dim(x2, c * CH_ROWS, (c + 1) * CH_ROWS), emb)
        for c in range(NCHUNK)
    ]
    out = _mlp_chunk_call(0)(es[0], *w_args)
    for c in range(1, NCHUNK):
        out = _mlp_chunk_call(c)(out, es[c], *w_args)
    return out.reshape(BATCH, SEQ, OUT_DIM)
